# Initial kernel scaffold; baseline (speedup 1.0000x reference)
import jax
import jax.numpy as jnp
from jax.experimental import pallas as pl


def kernel(x, edge_index, env_edge_attr, act_edge_attr, Ws0, Wn0, We0, b0, Ws1, Wn1, We1, b1, Ws2, Wn2, We2, b2, g0, beta0, g1, beta1):
    raise NotImplementedError("write your pallas kernel here")


# speedup vs baseline: 2.4653x; 2.4653x over previous
import functools

import jax
import jax.numpy as jnp
from jax import lax
from jax.experimental import pallas as pl
from jax.experimental.pallas import tpu as pltpu
from jax.experimental.pallas import tpu_sc as plsc

N = 10000
E = 320000
D = 128
DE = 16

NC = 2
NS = 16
NW = NC * NS
EPT = E // NW
CH = 80
NCHUNK = EPT // CH
ZR = 624
ZTAIL = N - NS * ZR


def _sc_msg_body(h_hbm, ep_hbm, src_hbm, dst_hbm, zero_hbm, out_hbm,
                 shared, srcb, dstb, rows, epb, gsem):
  c = lax.axis_index("c")
  s = lax.axis_index("s")
  wid = s * NC + c

  pltpu.sync_copy(zero_hbm.at[pl.ds(s * ZR, ZR)],
                  shared.at[pl.ds(s * ZR, ZR)])

  @pl.when(s == NS - 1)
  def _zero_tail():
    pltpu.sync_copy(zero_hbm.at[pl.ds(NS * ZR, ZTAIL)],
                    shared.at[pl.ds(NS * ZR, ZTAIL)])
  plsc.subcore_barrier()

  base = wid * EPT

  def chunk(k, carry):
    pltpu.sync_copy(src_hbm.at[wid, k], srcb)
    pltpu.sync_copy(dst_hbm.at[wid, k], dstb)
    pltpu.async_copy(h_hbm.at[srcb], rows, gsem).wait()
    pltpu.sync_copy(ep_hbm.at[pl.ds(base + k * CH, CH)], epb)

    def edge(e, carry2):
      for j in range(D // 16):
        sl = pl.ds(j * 16, 16)
        rows[e, sl] = jnp.maximum(rows[e, sl] + epb[e, sl], 0.0)
      return carry2

    lax.fori_loop(0, CH, edge, 0)
    pltpu.sync_copy(rows, shared.at[dstb], add=True)
    return carry

  lax.fori_loop(0, NCHUNK, chunk, 0)
  plsc.subcore_barrier()
  pltpu.sync_copy(shared.at[pl.ds(s * ZR, ZR)],
                  out_hbm.at[c, pl.ds(s * ZR, ZR)])

  @pl.when(s == NS - 1)
  def _dump_tail():
    pltpu.sync_copy(shared.at[pl.ds(NS * ZR, ZTAIL)],
                    out_hbm.at[c, pl.ds(NS * ZR, ZTAIL)])


_sc_msg = pl.kernel(
    _sc_msg_body,
    out_type=jax.ShapeDtypeStruct((NC, N, D), jnp.float32),
    mesh=plsc.VectorSubcoreMesh(core_axis_name="c", subcore_axis_name="s"),
    scratch_types=[
        pltpu.VMEM_SHARED((N, D), jnp.float32),
        pltpu.VMEM((CH,), jnp.int32),
        pltpu.VMEM((CH,), jnp.int32),
        pltpu.VMEM((CH, D), jnp.float32),
        pltpu.VMEM((CH, D), jnp.float32),
        pltpu.SemaphoreType.DMA,
    ],
)


def _eproj_body(attr_ref, we_ref, out_ref):
  out_ref[...] = jnp.dot(attr_ref[...], we_ref[...],
                         preferred_element_type=jnp.float32)


def _eproj(attr, We):
  BE = 4000
  return pl.pallas_call(
      _eproj_body,
      grid=(E // BE,),
      in_specs=[
          pl.BlockSpec((BE, DE), lambda i: (i, 0)),
          pl.BlockSpec((DE, D), lambda i: (0, 0)),
      ],
      out_specs=pl.BlockSpec((BE, D), lambda i: (i, 0)),
      out_shape=jax.ShapeDtypeStruct((E, D), jnp.float32),
  )(attr, We)


def _dense_body(h_ref, p_ref, ws_ref, wn_ref, b_ref, g_ref, beta_ref,
                out_ref, *, norm):
  agg = p_ref[0] + p_ref[1]
  y = (jnp.dot(h_ref[...], ws_ref[...], preferred_element_type=jnp.float32)
       + jnp.dot(agg, wn_ref[...], preferred_element_type=jnp.float32)
       + b_ref[...])
  if norm:
    mu = jnp.mean(y, axis=-1, keepdims=True)
    var = jnp.mean((y - mu) * (y - mu), axis=-1, keepdims=True)
    y = (y - mu) * lax.rsqrt(var + 1e-5) * g_ref[...] + beta_ref[...]
    y = jnp.maximum(y, 0.0)
  out_ref[...] = y


def _dense(h, parts, Ws, Wn, b, g, beta, norm):
  BN = 1000
  b2 = b.reshape(1, D)
  g2 = (g if g is not None else b).reshape(1, D)
  beta2 = (beta if beta is not None else b).reshape(1, D)
  return pl.pallas_call(
      functools.partial(_dense_body, norm=norm),
      grid=(N // BN,),
      in_specs=[
          pl.BlockSpec((BN, D), lambda i: (i, 0)),
          pl.BlockSpec((NC, BN, D), lambda i: (0, i, 0)),
          pl.BlockSpec((D, D), lambda i: (0, 0)),
          pl.BlockSpec((D, D), lambda i: (0, 0)),
          pl.BlockSpec((1, D), lambda i: (0, 0)),
          pl.BlockSpec((1, D), lambda i: (0, 0)),
          pl.BlockSpec((1, D), lambda i: (0, 0)),
      ],
      out_specs=pl.BlockSpec((BN, D), lambda i: (i, 0)),
      out_shape=jax.ShapeDtypeStruct((N, D), jnp.float32),
  )(h, parts, Ws, Wn, b2, g2, beta2)


def kernel(x, edge_index, env_edge_attr, act_edge_attr,
           Ws0, Wn0, We0, b0, Ws1, Wn1, We1, b1, Ws2, Wn2, We2, b2,
           g0, beta0, g1, beta1):
  src3 = edge_index[0].reshape(NW, NCHUNK, CH)
  dst3 = edge_index[1].reshape(NW, NCHUNK, CH)
  zeros = jnp.zeros((N, D), jnp.float32)

  ep0 = _eproj(env_edge_attr, We0)
  ep1 = _eproj(act_edge_attr, We1)
  ep2 = _eproj(act_edge_attr, We2)

  h = x
  parts = _sc_msg(h, ep0, src3, dst3, zeros)
  h = _dense(h, parts, Ws0, Wn0, b0, g0, beta0, norm=True)
  parts = _sc_msg(h, ep1, src3, dst3, zeros)
  h = _dense(h, parts, Ws1, Wn1, b1, g1, beta1, norm=True)
  parts = _sc_msg(h, ep2, src3, dst3, zeros)
  h = _dense(h, parts, Ws2, Wn2, b2, None, None, norm=False)
  return h


# speedup vs baseline: 4.0022x; 1.6234x over previous
import functools

import jax
import jax.numpy as jnp
from jax import lax
from jax.experimental import pallas as pl
from jax.experimental.pallas import tpu as pltpu
from jax.experimental.pallas import tpu_sc as plsc

N = 10000
E = 320000
D = 128
DE = 16

NC = 2
NS = 16
NW = NC * NS
EPT = E // NW
CH = 80
NCHUNK = EPT // CH
ZR = 624
ZTAIL = N - NS * ZR


def _sc_msg_body(h_hbm, ep_hbm, src_hbm, dst_hbm, zero_hbm, out_hbm,
                 shared, srcb0, srcb1, dstb0, dstb1,
                 rows0, rows1, epb0, epb1,
                 gsem0, gsem1, esem0, esem1):
  c = lax.axis_index("c")
  s = lax.axis_index("s")
  wid = s * NC + c
  srcb = (srcb0, srcb1)
  dstb = (dstb0, dstb1)
  rows = (rows0, rows1)
  epb = (epb0, epb1)
  gsem = (gsem0, gsem1)
  esem = (esem0, esem1)

  pltpu.sync_copy(zero_hbm.at[pl.ds(s * ZR, ZR)],
                  shared.at[pl.ds(s * ZR, ZR)])

  @pl.when(s == NS - 1)
  def _zero_tail():
    pltpu.sync_copy(zero_hbm.at[pl.ds(NS * ZR, ZTAIL)],
                    shared.at[pl.ds(NS * ZR, ZTAIL)])
  plsc.subcore_barrier()

  base = wid * EPT

  def fetch(chunk, b):
    pltpu.sync_copy(src_hbm.at[wid, chunk], srcb[b])
    pltpu.sync_copy(dst_hbm.at[wid, chunk], dstb[b])
    pltpu.async_copy(h_hbm.at[srcb[b]], rows[b], gsem[b])
    pltpu.async_copy(ep_hbm.at[pl.ds(base + chunk * CH, CH)], epb[b], esem[b])

  def consume(b):
    pltpu.make_async_copy(h_hbm.at[srcb[b]], rows[b], gsem[b]).wait()
    pltpu.make_async_copy(ep_hbm.at[pl.ds(0, CH)], epb[b], esem[b]).wait()

    def edge(e, carry2):
      for j in range(D // 16):
        sl = pl.ds(j * 16, 16)
        rows[b][e, sl] = jnp.maximum(rows[b][e, sl] + epb[b][e, sl], 0.0)
      return carry2

    lax.fori_loop(0, CH, edge, 0)
    pltpu.sync_copy(rows[b], shared.at[dstb[b]], add=True)

  fetch(0, 0)

  def pair(i, carry):
    k = i * 2
    for b in (0, 1):
      chunk = k + b

      @pl.when(chunk + 1 < NCHUNK)
      def _prefetch():
        fetch(chunk + 1, 1 - b)

      @pl.when(chunk < NCHUNK)
      def _consume():
        consume(b)
    return carry

  lax.fori_loop(0, (NCHUNK + 1) // 2, pair, 0)
  plsc.subcore_barrier()
  pltpu.sync_copy(shared.at[pl.ds(s * ZR, ZR)],
                  out_hbm.at[c, pl.ds(s * ZR, ZR)])

  @pl.when(s == NS - 1)
  def _dump_tail():
    pltpu.sync_copy(shared.at[pl.ds(NS * ZR, ZTAIL)],
                    out_hbm.at[c, pl.ds(NS * ZR, ZTAIL)])


_sc_msg = pl.kernel(
    _sc_msg_body,
    out_type=jax.ShapeDtypeStruct((NC, N, D), jnp.float32),
    mesh=plsc.VectorSubcoreMesh(core_axis_name="c", subcore_axis_name="s"),
    scratch_types=[
        pltpu.VMEM_SHARED((N, D), jnp.float32),
        pltpu.VMEM((CH,), jnp.int32),
        pltpu.VMEM((CH,), jnp.int32),
        pltpu.VMEM((CH,), jnp.int32),
        pltpu.VMEM((CH,), jnp.int32),
        pltpu.VMEM((CH, D), jnp.float32),
        pltpu.VMEM((CH, D), jnp.float32),
        pltpu.VMEM((CH, D), jnp.float32),
        pltpu.VMEM((CH, D), jnp.float32),
        pltpu.SemaphoreType.DMA,
        pltpu.SemaphoreType.DMA,
        pltpu.SemaphoreType.DMA,
        pltpu.SemaphoreType.DMA,
    ],
)


def _eproj_body(attr_ref, we_ref, out_ref):
  out_ref[...] = jnp.dot(attr_ref[...], we_ref[...],
                         preferred_element_type=jnp.float32)


def _eproj(attr, We):
  BE = 4000
  return pl.pallas_call(
      _eproj_body,
      grid=(E // BE,),
      in_specs=[
          pl.BlockSpec((BE, DE), lambda i: (i, 0)),
          pl.BlockSpec((DE, D), lambda i: (0, 0)),
      ],
      out_specs=pl.BlockSpec((BE, D), lambda i: (i, 0)),
      out_shape=jax.ShapeDtypeStruct((E, D), jnp.float32),
  )(attr, We)


def _dense_body(h_ref, p_ref, ws_ref, wn_ref, b_ref, g_ref, beta_ref,
                out_ref, *, norm):
  agg = p_ref[0] + p_ref[1]
  y = (jnp.dot(h_ref[...], ws_ref[...], preferred_element_type=jnp.float32)
       + jnp.dot(agg, wn_ref[...], preferred_element_type=jnp.float32)
       + b_ref[...])
  if norm:
    mu = jnp.mean(y, axis=-1, keepdims=True)
    var = jnp.mean((y - mu) * (y - mu), axis=-1, keepdims=True)
    y = (y - mu) * lax.rsqrt(var + 1e-5) * g_ref[...] + beta_ref[...]
    y = jnp.maximum(y, 0.0)
  out_ref[...] = y


def _dense(h, parts, Ws, Wn, b, g, beta, norm):
  BN = 1000
  b2 = b.reshape(1, D)
  g2 = (g if g is not None else b).reshape(1, D)
  beta2 = (beta if beta is not None else b).reshape(1, D)
  return pl.pallas_call(
      functools.partial(_dense_body, norm=norm),
      grid=(N // BN,),
      in_specs=[
          pl.BlockSpec((BN, D), lambda i: (i, 0)),
          pl.BlockSpec((NC, BN, D), lambda i: (0, i, 0)),
          pl.BlockSpec((D, D), lambda i: (0, 0)),
          pl.BlockSpec((D, D), lambda i: (0, 0)),
          pl.BlockSpec((1, D), lambda i: (0, 0)),
          pl.BlockSpec((1, D), lambda i: (0, 0)),
          pl.BlockSpec((1, D), lambda i: (0, 0)),
      ],
      out_specs=pl.BlockSpec((BN, D), lambda i: (i, 0)),
      out_shape=jax.ShapeDtypeStruct((N, D), jnp.float32),
  )(h, parts, Ws, Wn, b2, g2, beta2)


def kernel(x, edge_index, env_edge_attr, act_edge_attr,
           Ws0, Wn0, We0, b0, Ws1, Wn1, We1, b1, Ws2, Wn2, We2, b2,
           g0, beta0, g1, beta1):
  src3 = edge_index[0].reshape(NW, NCHUNK, CH)
  dst3 = edge_index[1].reshape(NW, NCHUNK, CH)
  zeros = jnp.zeros((N, D), jnp.float32)

  ep0 = _eproj(env_edge_attr, We0)
  ep1 = _eproj(act_edge_attr, We1)
  ep2 = _eproj(act_edge_attr, We2)

  h = x
  parts = _sc_msg(h, ep0, src3, dst3, zeros)
  h = _dense(h, parts, Ws0, Wn0, b0, g0, beta0, norm=True)
  parts = _sc_msg(h, ep1, src3, dst3, zeros)
  h = _dense(h, parts, Ws1, Wn1, b1, g1, beta1, norm=True)
  parts = _sc_msg(h, ep2, src3, dst3, zeros)
  h = _dense(h, parts, Ws2, Wn2, b2, None, None, norm=False)
  return h


# speedup vs baseline: 4.7029x; 1.1751x over previous
import functools

import jax
import jax.numpy as jnp
from jax import lax
from jax.experimental import pallas as pl
from jax.experimental.pallas import tpu as pltpu
from jax.experimental.pallas import tpu_sc as plsc

N = 10000
E = 320000
D = 128
DE = 16

NC = 2
NS = 16
NW = NC * NS
EPT = E // NW
CH = 80
NCHUNK = EPT // CH
ZR = 624
ZTAIL = N - NS * ZR


def _sc_msg_body(h_hbm, ep_hbm, src_hbm, dst_hbm, zero_hbm, out_hbm,
                 shared, srcb0, srcb1, srcb2, srcb3,
                 dstb0, dstb1, dstb2, dstb3,
                 rows0, rows1, epb0, epb1,
                 gsem0, gsem1, esem0, esem1,
                 isrc0, isrc1, isrc2, isrc3,
                 idst0, idst1, idst2, idst3):
  c = lax.axis_index("c")
  s = lax.axis_index("s")
  wid = s * NC + c
  srcb = (srcb0, srcb1, srcb2, srcb3)
  dstb = (dstb0, dstb1, dstb2, dstb3)
  rows = (rows0, rows1)
  epb = (epb0, epb1)
  gsem = (gsem0, gsem1)
  esem = (esem0, esem1)
  isrc = (isrc0, isrc1, isrc2, isrc3)
  idst = (idst0, idst1, idst2, idst3)

  pltpu.sync_copy(zero_hbm.at[pl.ds(s * ZR, ZR)],
                  shared.at[pl.ds(s * ZR, ZR)])

  @pl.when(s == NS - 1)
  def _zero_tail():
    pltpu.sync_copy(zero_hbm.at[pl.ds(NS * ZR, ZTAIL)],
                    shared.at[pl.ds(NS * ZR, ZTAIL)])
  plsc.subcore_barrier()

  base = wid * EPT

  def fetch_idx(chunk, q):
    pltpu.async_copy(src_hbm.at[wid, chunk], srcb[q], isrc[q])
    pltpu.async_copy(dst_hbm.at[wid, chunk], dstb[q], idst[q])

  def fetch_data(chunk, b, q):
    pltpu.make_async_copy(src_hbm.at[wid, chunk], srcb[q], isrc[q]).wait()
    pltpu.async_copy(h_hbm.at[srcb[q]], rows[b], gsem[b])
    pltpu.async_copy(ep_hbm.at[pl.ds(base + chunk * CH, CH)], epb[b], esem[b])

  fetch_idx(0, 0)
  fetch_idx(1, 1)
  fetch_data(0, 0, 0)

  def quad(i, carry):
    k = i * 4
    for u in range(4):
      chunk = k + u
      b = u % 2
      q = u

      @pl.when(chunk < NCHUNK)
      def _step():
        @pl.when(chunk + 2 < NCHUNK)
        def _next_idx():
          fetch_idx(chunk + 2, (q + 2) % 4)

        pltpu.make_async_copy(h_hbm.at[srcb[q]], rows[b], gsem[b]).wait()
        pltpu.make_async_copy(ep_hbm.at[pl.ds(0, CH)], epb[b], esem[b]).wait()

        @pl.when(chunk + 1 < NCHUNK)
        def _next_data():
          fetch_data(chunk + 1, 1 - b, (q + 1) % 4)

        def edge(e, carry2):
          for j in range(D // 16):
            sl = pl.ds(j * 16, 16)
            rows[b][e, sl] = jnp.maximum(rows[b][e, sl] + epb[b][e, sl], 0.0)
          return carry2

        lax.fori_loop(0, CH, edge, 0)
        pltpu.make_async_copy(dst_hbm.at[wid, chunk], dstb[q], idst[q]).wait()
        pltpu.sync_copy(rows[b], shared.at[dstb[q]], add=True)
    return carry

  lax.fori_loop(0, (NCHUNK + 3) // 4, quad, 0)
  plsc.subcore_barrier()
  pltpu.sync_copy(shared.at[pl.ds(s * ZR, ZR)],
                  out_hbm.at[c, pl.ds(s * ZR, ZR)])

  @pl.when(s == NS - 1)
  def _dump_tail():
    pltpu.sync_copy(shared.at[pl.ds(NS * ZR, ZTAIL)],
                    out_hbm.at[c, pl.ds(NS * ZR, ZTAIL)])


_sc_msg = pl.kernel(
    _sc_msg_body,
    out_type=jax.ShapeDtypeStruct((NC, N, D), jnp.float32),
    mesh=plsc.VectorSubcoreMesh(core_axis_name="c", subcore_axis_name="s"),
    scratch_types=[
        pltpu.VMEM_SHARED((N, D), jnp.float32),
        pltpu.VMEM((CH,), jnp.int32),
        pltpu.VMEM((CH,), jnp.int32),
        pltpu.VMEM((CH,), jnp.int32),
        pltpu.VMEM((CH,), jnp.int32),
        pltpu.VMEM((CH,), jnp.int32),
        pltpu.VMEM((CH,), jnp.int32),
        pltpu.VMEM((CH,), jnp.int32),
        pltpu.VMEM((CH,), jnp.int32),
        pltpu.VMEM((CH, D), jnp.float32),
        pltpu.VMEM((CH, D), jnp.float32),
        pltpu.VMEM((CH, D), jnp.float32),
        pltpu.VMEM((CH, D), jnp.float32),
        pltpu.SemaphoreType.DMA,
        pltpu.SemaphoreType.DMA,
        pltpu.SemaphoreType.DMA,
        pltpu.SemaphoreType.DMA,
        pltpu.SemaphoreType.DMA,
        pltpu.SemaphoreType.DMA,
        pltpu.SemaphoreType.DMA,
        pltpu.SemaphoreType.DMA,
        pltpu.SemaphoreType.DMA,
        pltpu.SemaphoreType.DMA,
        pltpu.SemaphoreType.DMA,
        pltpu.SemaphoreType.DMA,
    ],
)


def _eproj_body(attr_ref, we_ref, out_ref):
  out_ref[...] = jnp.dot(attr_ref[...], we_ref[...],
                         preferred_element_type=jnp.float32)


def _eproj(attr, We):
  BE = 4000
  return pl.pallas_call(
      _eproj_body,
      grid=(E // BE,),
      in_specs=[
          pl.BlockSpec((BE, DE), lambda i: (i, 0)),
          pl.BlockSpec((DE, D), lambda i: (0, 0)),
      ],
      out_specs=pl.BlockSpec((BE, D), lambda i: (i, 0)),
      out_shape=jax.ShapeDtypeStruct((E, D), jnp.float32),
  )(attr, We)


def _dense_body(h_ref, p_ref, ws_ref, wn_ref, b_ref, g_ref, beta_ref,
                out_ref, *, norm):
  agg = p_ref[0] + p_ref[1]
  y = (jnp.dot(h_ref[...], ws_ref[...], preferred_element_type=jnp.float32)
       + jnp.dot(agg, wn_ref[...], preferred_element_type=jnp.float32)
       + b_ref[...])
  if norm:
    mu = jnp.mean(y, axis=-1, keepdims=True)
    var = jnp.mean((y - mu) * (y - mu), axis=-1, keepdims=True)
    y = (y - mu) * lax.rsqrt(var + 1e-5) * g_ref[...] + beta_ref[...]
    y = jnp.maximum(y, 0.0)
  out_ref[...] = y


def _dense(h, parts, Ws, Wn, b, g, beta, norm):
  BN = 1000
  b2 = b.reshape(1, D)
  g2 = (g if g is not None else b).reshape(1, D)
  beta2 = (beta if beta is not None else b).reshape(1, D)
  return pl.pallas_call(
      functools.partial(_dense_body, norm=norm),
      grid=(N // BN,),
      in_specs=[
          pl.BlockSpec((BN, D), lambda i: (i, 0)),
          pl.BlockSpec((NC, BN, D), lambda i: (0, i, 0)),
          pl.BlockSpec((D, D), lambda i: (0, 0)),
          pl.BlockSpec((D, D), lambda i: (0, 0)),
          pl.BlockSpec((1, D), lambda i: (0, 0)),
          pl.BlockSpec((1, D), lambda i: (0, 0)),
          pl.BlockSpec((1, D), lambda i: (0, 0)),
      ],
      out_specs=pl.BlockSpec((BN, D), lambda i: (i, 0)),
      out_shape=jax.ShapeDtypeStruct((N, D), jnp.float32),
  )(h, parts, Ws, Wn, b2, g2, beta2)


def kernel(x, edge_index, env_edge_attr, act_edge_attr,
           Ws0, Wn0, We0, b0, Ws1, Wn1, We1, b1, Ws2, Wn2, We2, b2,
           g0, beta0, g1, beta1):
  src3 = edge_index[0].reshape(NW, NCHUNK, CH)
  dst3 = edge_index[1].reshape(NW, NCHUNK, CH)
  zeros = jnp.zeros((N, D), jnp.float32)

  ep0 = _eproj(env_edge_attr, We0)
  ep1 = _eproj(act_edge_attr, We1)
  ep2 = _eproj(act_edge_attr, We2)

  h = x
  parts = _sc_msg(h, ep0, src3, dst3, zeros)
  h = _dense(h, parts, Ws0, Wn0, b0, g0, beta0, norm=True)
  parts = _sc_msg(h, ep1, src3, dst3, zeros)
  h = _dense(h, parts, Ws1, Wn1, b1, g1, beta1, norm=True)
  parts = _sc_msg(h, ep2, src3, dst3, zeros)
  h = _dense(h, parts, Ws2, Wn2, b2, None, None, norm=False)
  return h
